# baseline (speedup 1.0000x reference)
import functools

import jax
import jax.numpy as jnp
from jax import lax
from jax.experimental import pallas as pl
from jax.experimental.pallas import tpu as pltpu
from jax.experimental.pallas import tpu_sc as plsc

N, T, C, P = 16, 2048, 500, 100
L = 16
PG = 7
PPAD = PG * L
TB = 64
RPC = TB * C // 128
LBLK = 128
NPC = 8
THALF = T // 2
NCHUNK = THALF // TB
NBLK = THALF // LBLK

_MESH = plsc.VectorSubcoreMesh(
    core_axis_name="c", subcore_axis_name="s", num_cores=2, num_subcores=16
)
_PARAMS = pltpu.CompilerParams(
    use_tc_tiling_on_sc=False, needs_layout_passes=False
)


def _fused_body(nnet_hbm, labels_hbm, wers_hbm, out_hbm,
                rows2_v, labs2_v, acc_v, big_v, wers_v, out_v, shared_v,
                sem_r0, sem_r1, sem_lab):
    c = lax.axis_index("c")
    s = lax.axis_index("s")
    n = c * NPC + s // 2
    half = s % 2
    t0base = half * THALF
    blkbase = half * NBLK
    sem_r = (sem_r0, sem_r1)

    iota = lax.iota(jnp.int32, L)
    rowvecs = []
    for j in range(PG):
        rv = iota + (j * L)
        if (j + 1) * L > P:
            rv = jnp.minimum(rv, P - 1)
        rowvecs.append(rv)

    def r_copy(g, slot):
        t0 = t0base + g * TB
        return pltpu.make_async_copy(
            nnet_hbm.at[pl.ds((n * T + t0) * C, TB * C)],
            rows2_v.at[slot], sem_r[slot])

    def l_start(b):
        pltpu.make_async_copy(
            labels_hbm.at[n, :, blkbase + b, :],
            labs2_v.at[b % 2], sem_lab).start()

    def l_wait():
        pltpu.make_async_copy(
            labels_hbm.at[n, :, blkbase, :],
            labs2_v.at[0], sem_lab).wait()

    def compute(slot, g, accs):
        rslot = rows2_v.at[slot]
        bslot_v = jnp.full((L,), (g // 2) % 2, jnp.int32)
        col0 = (g % 2) * TB

        def fbody(q, accs):
            new = list(accs)
            for u in range(4):
                t = q * 4 + u
                tlv = jnp.full((L,), col0 + t, jnp.int32)
                fv = jnp.full((L,), t * C, jnp.int32)
                for j in range(PG):
                    lab = plsc.load_gather(
                        labs2_v, [bslot_v, rowvecs[j], tlv])
                    gv = plsc.load_gather(rslot, [lab + fv])
                    new[j] = new[j] + gv
            return tuple(new)

        return lax.fori_loop(0, TB // 4, fbody, accs)

    l_start(0)
    r_copy(0, 0).start()
    r_copy(1, 1).start()

    def body(k, accs):
        g = 2 * k
        l_wait()
        l_start(k + 1)
        r_copy(g, 0).wait()
        accs = compute(0, g, accs)
        r_copy(jnp.minimum(g + 2, NCHUNK - 1), 0).start()
        r_copy(g + 1, 1).wait()
        accs = compute(1, g + 1, accs)
        r_copy(jnp.minimum(g + 3, NCHUNK - 1), 1).start()
        return accs

    zero = jnp.zeros((L,), jnp.float32)
    accs = lax.fori_loop(0, NCHUNK // 2 - 1, body, (zero,) * PG)
    l_wait()
    r_copy(NCHUNK - 2, 0).wait()
    accs = compute(0, NCHUNK - 2, accs)
    r_copy(NCHUNK - 1, 1).wait()
    accs = compute(1, NCHUNK - 1, accs)

    for j in range(PG):
        acc_v[pl.ds(j * L, L)] = accs[j]

    pltpu.sync_copy(acc_v, shared_v.at[s])
    plsc.subcore_barrier()

    @pl.when(s == 0)
    def _():
        pltpu.sync_copy(shared_v, big_v)
        pltpu.sync_copy(
            wers_hbm.at[pl.ds(c * (NPC * PPAD), NPC * PPAD)], wers_v)
        lane = lax.iota(jnp.int32, L)
        neg = jnp.full((L,), -3.0e38, jnp.float32)
        total = jnp.zeros((L,), jnp.float32)
        for m in range(NPC):
            svecs = []
            for j in range(PG):
                sv = (big_v[2 * m, pl.ds(j * L, L)]
                      + big_v[2 * m + 1, pl.ds(j * L, L)])
                msk = (lane + (j * L)) < P
                svecs.append(jnp.where(msk, sv, neg))
            mvec = svecs[0]
            for j in range(1, PG):
                mvec = jnp.maximum(mvec, svecs[j])
            mmax = jnp.max(mvec)
            den = jnp.zeros((L,), jnp.float32)
            num = jnp.zeros((L,), jnp.float32)
            for j in range(PG):
                e = jnp.exp(svecs[j] - mmax)
                den = den + e
                num = num + e * wers_v[pl.ds(m * PPAD + j * L, L)]
            numsum = jnp.full((L,), jnp.sum(num), jnp.float32)
            densum = jnp.full((L,), jnp.sum(den), jnp.float32)
            total = total + numsum / densum
        out_v[...] = total * (1.0 / (N * P))
        pltpu.sync_copy(out_v, out_hbm.at[pl.ds(c * L, L)])


_fused = functools.partial(
    pl.kernel,
    out_type=jax.ShapeDtypeStruct((2 * L,), jnp.float32),
    mesh=_MESH,
    compiler_params=_PARAMS,
    scratch_types=[
        pltpu.VMEM((2, TB * C), jnp.float32),
        pltpu.VMEM((2, P, LBLK), jnp.int32),
        pltpu.VMEM((PPAD,), jnp.float32),
        pltpu.VMEM((2 * NPC, PPAD), jnp.float32),
        pltpu.VMEM((NPC * PPAD,), jnp.float32),
        pltpu.VMEM((L,), jnp.float32),
        pltpu.VMEM_SHARED((2 * NPC, PPAD), jnp.float32),
        pltpu.SemaphoreType.DMA,
        pltpu.SemaphoreType.DMA,
        pltpu.SemaphoreType.DMA,
    ],
)(_fused_body)


def kernel(nnet_output, path_labels, wers):
    nnet1 = nnet_output.reshape(-1)
    labels4 = path_labels.astype(jnp.int32).reshape(N, P, T // 128, 128)
    wers_f = jnp.pad(
        wers.astype(jnp.float32), ((0, 0), (0, PPAD - P))).reshape(-1)
    out = _fused(nnet1, labels4, wers_f)
    return out[0] + out[L]


# speedup vs baseline: 1.0511x; 1.0511x over previous
import functools

import jax
import jax.numpy as jnp
from jax import lax
from jax.experimental import pallas as pl
from jax.experimental.pallas import tpu as pltpu
from jax.experimental.pallas import tpu_sc as plsc

N, T, C, P = 16, 2048, 500, 100
L = 16
PG = 7
PPAD = PG * L
TB = 16
NPC = 8
THALF = T // 2
NCHUNK = THALF // TB

_MESH = plsc.VectorSubcoreMesh(
    core_axis_name="c", subcore_axis_name="s", num_cores=2, num_subcores=16
)
_PARAMS = pltpu.CompilerParams(
    use_tc_tiling_on_sc=False, needs_layout_passes=False
)


def _fused_body(nnet_hbm, labels_hbm, wers_hbm, out_hbm,
                rows2_v, labs2_v, acc_v, big_v, wers_v, out_v, shared_v,
                sem_r0, sem_r1, sem_l0, sem_l1):
    c = lax.axis_index("c")
    s = lax.axis_index("s")
    n = c * NPC + s // 2
    half = s % 2
    t0base = half * THALF
    sem_r = (sem_r0, sem_r1)
    sem_l = (sem_l0, sem_l1)

    zero16i = jnp.zeros((L,), jnp.int32)
    for slot in range(2):
        for r in range(P, PPAD):
            labs2_v[slot, r, :] = zero16i

    def dma_pair(chunk, slot):
        t0 = t0base + chunk * TB
        rcp = pltpu.make_async_copy(
            nnet_hbm.at[pl.ds((n * T + t0) * C, TB * C)],
            rows2_v.at[slot], sem_r[slot])
        lcp = pltpu.make_async_copy(
            labels_hbm.at[n, :, pl.ds(t0, TB)],
            labs2_v.at[slot, pl.ds(0, P), :], sem_l[slot])
        return rcp, lcp

    def start(chunk, slot):
        rcp, lcp = dma_pair(chunk, slot)
        rcp.start()
        lcp.start()

    def wait(chunk, slot):
        rcp, lcp = dma_pair(chunk, slot)
        rcp.wait()
        lcp.wait()

    def compute(slot, accs):
        new = list(accs)
        rslot = rows2_v.at[slot]
        lslot = labs2_v.at[slot]
        for t in range(TB):
            tv = jnp.full((L,), t, jnp.int32)
            tc = jnp.full((L,), t * C, jnp.int32)
            for j in range(PG):
                rows_j = lax.iota(jnp.int32, L) + (j * L)
                lab = plsc.load_gather(lslot, [rows_j, tv])
                g = plsc.load_gather(rslot, [lab + tc])
                new[j] = new[j] + g
        return tuple(new)

    start(0, 0)
    start(1, 1)

    def body(k, accs):
        c0 = 2 * k
        wait(c0, 0)
        accs = compute(0, accs)
        start(c0 + 2, 0)
        wait(c0 + 1, 1)
        accs = compute(1, accs)
        start(c0 + 3, 1)
        return accs

    zero = jnp.zeros((L,), jnp.float32)
    accs = lax.fori_loop(0, NCHUNK // 2 - 1, body, (zero,) * PG)
    wait(NCHUNK - 2, 0)
    accs = compute(0, accs)
    wait(NCHUNK - 1, 1)
    accs = compute(1, accs)

    for j in range(PG):
        acc_v[pl.ds(j * L, L)] = accs[j]

    pltpu.sync_copy(acc_v, shared_v.at[s])
    plsc.subcore_barrier()

    @pl.when(s == 0)
    def _():
        pltpu.sync_copy(shared_v, big_v)
        pltpu.sync_copy(
            wers_hbm.at[pl.ds(c * (NPC * PPAD), NPC * PPAD)], wers_v)
        lane = lax.iota(jnp.int32, L)
        neg = jnp.full((L,), -3.0e38, jnp.float32)
        total = jnp.zeros((L,), jnp.float32)
        for m in range(NPC):
            svecs = []
            for j in range(PG):
                sv = (big_v[2 * m, pl.ds(j * L, L)]
                      + big_v[2 * m + 1, pl.ds(j * L, L)])
                msk = (lane + (j * L)) < P
                svecs.append(jnp.where(msk, sv, neg))
            mvec = svecs[0]
            for j in range(1, PG):
                mvec = jnp.maximum(mvec, svecs[j])
            mmax = jnp.max(mvec)
            den = jnp.zeros((L,), jnp.float32)
            num = jnp.zeros((L,), jnp.float32)
            for j in range(PG):
                e = jnp.exp(svecs[j] - mmax)
                den = den + e
                num = num + e * wers_v[pl.ds(m * PPAD + j * L, L)]
            numsum = jnp.full((L,), jnp.sum(num), jnp.float32)
            densum = jnp.full((L,), jnp.sum(den), jnp.float32)
            total = total + numsum / densum
        out_v[...] = total * (1.0 / (N * P))
        pltpu.sync_copy(out_v, out_hbm.at[pl.ds(c * L, L)])


_fused = functools.partial(
    pl.kernel,
    out_type=jax.ShapeDtypeStruct((2 * L,), jnp.float32),
    mesh=_MESH,
    compiler_params=_PARAMS,
    scratch_types=[
        pltpu.VMEM((2, TB * C), jnp.float32),
        pltpu.VMEM((2, PPAD, TB), jnp.int32),
        pltpu.VMEM((PPAD,), jnp.float32),
        pltpu.VMEM((2 * NPC, PPAD), jnp.float32),
        pltpu.VMEM((NPC * PPAD,), jnp.float32),
        pltpu.VMEM((L,), jnp.float32),
        pltpu.VMEM_SHARED((2 * NPC, PPAD), jnp.float32),
        pltpu.SemaphoreType.DMA,
        pltpu.SemaphoreType.DMA,
        pltpu.SemaphoreType.DMA,
        pltpu.SemaphoreType.DMA,
    ],
)(_fused_body)


def kernel(nnet_output, path_labels, wers):
    nnet1 = nnet_output.reshape(-1)
    labels_i = path_labels.astype(jnp.int32)
    wers_f = jnp.pad(
        wers.astype(jnp.float32), ((0, 0), (0, PPAD - P))).reshape(-1)
    out = _fused(nnet1, labels_i, wers_f)
    return out[0] + out[L]
